# Initial kernel scaffold; baseline (speedup 1.0000x reference)
import jax
import jax.numpy as jnp
from jax.experimental import pallas as pl


def kernel(x, embed):
    raise NotImplementedError("write your pallas kernel here")


# speedup vs baseline: 1.2704x; 1.2704x over previous
import functools

import jax
import jax.numpy as jnp
from jax.experimental import pallas as pl
from jax.experimental.pallas import tpu as pltpu

DIM = 32
K = 512


def _vq_block(x_ref, embed_ref, q_ref, ind_ref):
    xb = x_ref[...]
    G, T, _ = xb.shape
    N = G * T
    xf = xb.reshape(N, DIM)
    e = embed_ref[...]
    xe = jax.lax.dot_general(
        xf, e, (((1,), (1,)), ((), ())),
        preferred_element_type=jnp.float32,
    )
    x2 = jnp.sum(xf * xf, axis=1, keepdims=True)
    e2 = jnp.sum(e * e, axis=1)[None, :]
    dist = -(x2 - 2.0 * xe + e2)
    ind = jnp.argmax(dist, axis=-1).astype(jnp.int32)
    onehot = (
        ind[:, None] == jax.lax.broadcasted_iota(jnp.int32, (N, K), 1)
    ).astype(jnp.float32)
    q = jax.lax.dot_general(
        onehot, e, (((1,), (0,)), ((), ())),
        preferred_element_type=jnp.float32,
        precision=jax.lax.Precision.HIGHEST,
    )
    ind_ref[...] = ind.reshape(G, 1, T)
    q_ref[...] = q.reshape(G, T, DIM)


@functools.partial(jax.jit, static_argnames=("g",))
def _vq(x, embed, g=2):
    B, T, D = x.shape
    grid = (B // g,)
    q, ind = pl.pallas_call(
        _vq_block,
        grid=grid,
        in_specs=[
            pl.BlockSpec((g, T, D), lambda i: (i, 0, 0)),
            pl.BlockSpec((K, D), lambda i: (0, 0)),
        ],
        out_specs=[
            pl.BlockSpec((g, T, D), lambda i: (i, 0, 0)),
            pl.BlockSpec((g, 1, T), lambda i: (i, 0, 0)),
        ],
        out_shape=[
            jax.ShapeDtypeStruct((B, T, D), jnp.float32),
            jax.ShapeDtypeStruct((B, 1, T), jnp.int32),
        ],
    )(x, embed)
    return q, ind.reshape(B, T)


def kernel(x, embed):
    quantize, embed_ind = _vq(x, embed)
    return (quantize, embed_ind)


# speedup vs baseline: 1.7835x; 1.4038x over previous
import functools

import jax
import jax.numpy as jnp
from jax.experimental import pallas as pl
from jax.experimental.pallas import tpu as pltpu

DIM = 32
K = 512


def _vq_block(x_ref, embed_ref, q_ref, ind_ref):
    xb = x_ref[...]
    G, T, _ = xb.shape
    N = G * T
    xf = xb.reshape(N, DIM)
    e = embed_ref[...]
    xe = jax.lax.dot_general(
        xf, e, (((1,), (1,)), ((), ())),
        preferred_element_type=jnp.float32,
    )
    x2 = jnp.sum(xf * xf, axis=1, keepdims=True)
    e2 = jnp.sum(e * e, axis=1)[None, :]
    dist = -(x2 - 2.0 * xe + e2)
    ind = jnp.argmax(dist, axis=-1).astype(jnp.int32)
    onehot = (
        ind[:, None] == jax.lax.broadcasted_iota(jnp.int32, (N, K), 1)
    ).astype(jnp.bfloat16)
    e_hi = e.astype(jnp.bfloat16)
    e_lo = (e - e_hi.astype(jnp.float32)).astype(jnp.bfloat16)
    q = jax.lax.dot_general(
        onehot, e_hi, (((1,), (0,)), ((), ())),
        preferred_element_type=jnp.float32,
    ) + jax.lax.dot_general(
        onehot, e_lo, (((1,), (0,)), ((), ())),
        preferred_element_type=jnp.float32,
    )
    ind_ref[...] = ind.reshape(G, 1, T)
    q_ref[...] = q.reshape(G, T, DIM)


@functools.partial(jax.jit, static_argnames=("g",))
def _vq(x, embed, g=2):
    B, T, D = x.shape
    grid = (B // g,)
    q, ind = pl.pallas_call(
        _vq_block,
        grid=grid,
        in_specs=[
            pl.BlockSpec((g, T, D), lambda i: (i, 0, 0)),
            pl.BlockSpec((K, D), lambda i: (0, 0)),
        ],
        out_specs=[
            pl.BlockSpec((g, T, D), lambda i: (i, 0, 0)),
            pl.BlockSpec((g, 1, T), lambda i: (i, 0, 0)),
        ],
        out_shape=[
            jax.ShapeDtypeStruct((B, T, D), jnp.float32),
            jax.ShapeDtypeStruct((B, 1, T), jnp.int32),
        ],
    )(x, embed)
    return q, ind.reshape(B, T)


def kernel(x, embed):
    quantize, embed_ind = _vq(x, embed)
    return (quantize, embed_ind)


# speedup vs baseline: 1.8432x; 1.0335x over previous
import functools

import jax
import jax.numpy as jnp
from jax.experimental import pallas as pl
from jax.experimental.pallas import tpu as pltpu

DIM = 32
K = 512


def _vq_block(x_ref, embed_ref, q_ref, ind_ref):
    xb = x_ref[...]
    G, T, _ = xb.shape
    N = G * T
    xf = xb.reshape(N, DIM)
    e = embed_ref[...]
    xe = jax.lax.dot_general(
        xf, e, (((1,), (1,)), ((), ())),
        preferred_element_type=jnp.float32,
    )
    x2 = jnp.sum(xf * xf, axis=1, keepdims=True)
    e2 = jnp.sum(e * e, axis=1)[None, :]
    dist = -(x2 - 2.0 * xe + e2)
    ind = jnp.argmax(dist, axis=-1).astype(jnp.int32)
    onehot = (
        ind[:, None] == jax.lax.broadcasted_iota(jnp.int32, (N, K), 1)
    ).astype(jnp.bfloat16)
    e_hi = e.astype(jnp.bfloat16)
    e_lo = (e - e_hi.astype(jnp.float32)).astype(jnp.bfloat16)
    q = jax.lax.dot_general(
        onehot, e_hi, (((1,), (0,)), ((), ())),
        preferred_element_type=jnp.float32,
    ) + jax.lax.dot_general(
        onehot, e_lo, (((1,), (0,)), ((), ())),
        preferred_element_type=jnp.float32,
    )
    ind_ref[...] = ind.reshape(G, 1, T)
    q_ref[...] = q.reshape(G, T, DIM)


@functools.partial(jax.jit, static_argnames=("g",))
def _vq(x, embed, g=4):
    B, T, D = x.shape
    grid = (B // g,)
    q, ind = pl.pallas_call(
        _vq_block,
        grid=grid,
        in_specs=[
            pl.BlockSpec((g, T, D), lambda i: (i, 0, 0)),
            pl.BlockSpec((K, D), lambda i: (0, 0)),
        ],
        out_specs=[
            pl.BlockSpec((g, T, D), lambda i: (i, 0, 0)),
            pl.BlockSpec((g, 1, T), lambda i: (i, 0, 0)),
        ],
        out_shape=[
            jax.ShapeDtypeStruct((B, T, D), jnp.float32),
            jax.ShapeDtypeStruct((B, 1, T), jnp.int32),
        ],
    )(x, embed)
    return q, ind.reshape(B, T)


def kernel(x, embed):
    quantize, embed_ind = _vq(x, embed)
    return (quantize, embed_ind)


# speedup vs baseline: 1.8555x; 1.0066x over previous
import functools

import jax
import jax.numpy as jnp
from jax.experimental import pallas as pl
from jax.experimental.pallas import tpu as pltpu

DIM = 32
K = 512


def _vq_block(x_ref, embed_ref, q_ref, ind_ref):
    xb = x_ref[...]
    G, T, _ = xb.shape
    N = G * T
    xf = xb.reshape(N, DIM)
    e = embed_ref[...]
    xe = jax.lax.dot_general(
        xf, e, (((1,), (1,)), ((), ())),
        preferred_element_type=jnp.float32,
    )
    x2 = jnp.sum(xf * xf, axis=1, keepdims=True)
    e2 = jnp.sum(e * e, axis=1)[None, :]
    dist = -(x2 - 2.0 * xe + e2)
    ind = jnp.argmax(dist, axis=-1).astype(jnp.int32)
    onehot = (
        ind[:, None] == jax.lax.broadcasted_iota(jnp.int32, (N, K), 1)
    ).astype(jnp.bfloat16)
    e_hi = e.astype(jnp.bfloat16)
    e_lo = (e - e_hi.astype(jnp.float32)).astype(jnp.bfloat16)
    q = jax.lax.dot_general(
        onehot, e_hi, (((1,), (0,)), ((), ())),
        preferred_element_type=jnp.float32,
    ) + jax.lax.dot_general(
        onehot, e_lo, (((1,), (0,)), ((), ())),
        preferred_element_type=jnp.float32,
    )
    ind_ref[...] = ind.reshape(G, 1, T)
    q_ref[...] = q.reshape(G, T, DIM)


@functools.partial(jax.jit, static_argnames=("g",))
def _vq(x, embed, g=8):
    B, T, D = x.shape
    grid = (B // g,)
    q, ind = pl.pallas_call(
        _vq_block,
        grid=grid,
        in_specs=[
            pl.BlockSpec((g, T, D), lambda i: (i, 0, 0)),
            pl.BlockSpec((K, D), lambda i: (0, 0)),
        ],
        out_specs=[
            pl.BlockSpec((g, T, D), lambda i: (i, 0, 0)),
            pl.BlockSpec((g, 1, T), lambda i: (i, 0, 0)),
        ],
        out_shape=[
            jax.ShapeDtypeStruct((B, T, D), jnp.float32),
            jax.ShapeDtypeStruct((B, 1, T), jnp.int32),
        ],
    )(x, embed)
    return q, ind.reshape(B, T)


def kernel(x, embed):
    quantize, embed_ind = _vq(x, embed)
    return (quantize, embed_ind)


# speedup vs baseline: 2.3027x; 1.2411x over previous
import functools

import jax
import jax.numpy as jnp
from jax.experimental import pallas as pl
from jax.experimental.pallas import tpu as pltpu

DIM = 32
K = 512


def _vq_block(x_ref, embed_ref, q_ref, ind_ref):
    xb = x_ref[...]
    G, T, _ = xb.shape
    N = G * T
    xf = xb.reshape(N, DIM)
    e = embed_ref[...]
    xe = jax.lax.dot_general(
        xf, e, (((1,), (1,)), ((), ())),
        preferred_element_type=jnp.float32,
    )
    x2 = jnp.sum(xf * xf, axis=1, keepdims=True)
    e2 = jnp.sum(e * e, axis=1)[None, :]
    b = x2 - 2.0 * xe + e2
    ind = jnp.argmin(b, axis=-1).astype(jnp.int32)
    onehot = (
        ind[:, None] == jax.lax.broadcasted_iota(jnp.int32, (N, K), 1)
    ).astype(jnp.bfloat16)
    e_hi = e.astype(jnp.bfloat16)
    e_lo = (e - e_hi.astype(jnp.float32)).astype(jnp.bfloat16)
    e_cat = jnp.concatenate([e_hi, e_lo], axis=1)
    q2 = jax.lax.dot_general(
        onehot, e_cat, (((1,), (0,)), ((), ())),
        preferred_element_type=jnp.float32,
    )
    q = q2[:, :DIM] + q2[:, DIM:]
    ind_ref[...] = ind.reshape(G, 1, T)
    q_ref[...] = q.reshape(G, T, DIM)


@functools.partial(jax.jit, static_argnames=("g",))
def _vq(x, embed, g=8):
    B, T, D = x.shape
    grid = (B // g,)
    q, ind = pl.pallas_call(
        _vq_block,
        grid=grid,
        in_specs=[
            pl.BlockSpec((g, T, D), lambda i: (i, 0, 0)),
            pl.BlockSpec((K, D), lambda i: (0, 0)),
        ],
        out_specs=[
            pl.BlockSpec((g, T, D), lambda i: (i, 0, 0)),
            pl.BlockSpec((g, 1, T), lambda i: (i, 0, 0)),
        ],
        out_shape=[
            jax.ShapeDtypeStruct((B, T, D), jnp.float32),
            jax.ShapeDtypeStruct((B, 1, T), jnp.int32),
        ],
    )(x, embed)
    return q, ind.reshape(B, T)


def kernel(x, embed):
    quantize, embed_ind = _vq(x, embed)
    return (quantize, embed_ind)


# speedup vs baseline: 2.5057x; 1.0882x over previous
import functools

import jax
import jax.numpy as jnp
from jax.experimental import pallas as pl
from jax.experimental.pallas import tpu as pltpu

DIM = 32
K = 512


def _vq_block(x_ref, embed_ref, q_ref, ind_ref):
    xb = x_ref[...]
    G, T, _ = xb.shape
    N = G * T
    xf = xb.reshape(N, DIM)
    e = embed_ref[...]
    xe = jax.lax.dot_general(
        xf, e, (((1,), (1,)), ((), ())),
        preferred_element_type=jnp.float32,
    )
    x2 = jnp.sum(xf * xf, axis=1, keepdims=True)
    e2 = jnp.sum(e * e, axis=1)[None, :]
    b = x2 - 2.0 * xe + e2
    m_row = jnp.min(b, axis=-1, keepdims=True)
    onehot = (b == m_row).astype(jnp.bfloat16)
    e_hi = e.astype(jnp.bfloat16)
    e_lo = (e - e_hi.astype(jnp.float32)).astype(jnp.bfloat16)
    k_iota = jax.lax.broadcasted_iota(jnp.int32, (K, 1), 0)
    i_hi = (k_iota // 16).astype(jnp.bfloat16)
    i_lo = (k_iota % 16).astype(jnp.bfloat16)
    e_cat = jnp.concatenate([e_hi, e_lo, i_hi, i_lo], axis=1)
    q2 = jax.lax.dot_general(
        onehot, e_cat, (((1,), (0,)), ((), ())),
        preferred_element_type=jnp.float32,
    )
    q = q2[:, :DIM] + q2[:, DIM:2 * DIM]
    ind = (
        16.0 * q2[:, 2 * DIM] + q2[:, 2 * DIM + 1]
    ).astype(jnp.int32)
    ind_ref[...] = ind.reshape(G, 1, T)
    q_ref[...] = q.reshape(G, T, DIM)


@functools.partial(jax.jit, static_argnames=("g",))
def _vq(x, embed, g=8):
    B, T, D = x.shape
    grid = (B // g,)
    q, ind = pl.pallas_call(
        _vq_block,
        grid=grid,
        in_specs=[
            pl.BlockSpec((g, T, D), lambda i: (i, 0, 0)),
            pl.BlockSpec((K, D), lambda i: (0, 0)),
        ],
        out_specs=[
            pl.BlockSpec((g, T, D), lambda i: (i, 0, 0)),
            pl.BlockSpec((g, 1, T), lambda i: (i, 0, 0)),
        ],
        out_shape=[
            jax.ShapeDtypeStruct((B, T, D), jnp.float32),
            jax.ShapeDtypeStruct((B, 1, T), jnp.int32),
        ],
    )(x, embed)
    return q, ind.reshape(B, T)


def kernel(x, embed):
    quantize, embed_ind = _vq(x, embed)
    return (quantize, embed_ind)
